# baseline (speedup 1.0000x reference)
import functools

import jax
import jax.numpy as jnp
from jax import lax
from jax.experimental import pallas as pl
from jax.experimental.pallas import tpu as pltpu
from jax.experimental.pallas import tpu_sc as plsc

N_NODES_C = 10000
D_EMB_C = 128
E_C = 320000
CHUNK = 128
N_CHUNKS = E_C // CHUNK
L = 16
DF = 16


def _sc_kernel_body(emb_hbm, src_id_hbm, dst_id_hbm, featsp_hbm, wvec_hbm,
                    out_hbm, idx_s, idx_d, src_rows, dst_rows, feats_v,
                    out_v, wv, sem_s, sem_d):
    nc = plsc.get_sparse_core_info().num_cores
    wid = lax.axis_index("s") * nc + lax.axis_index("c")
    n_workers = 32
    n_groups = CHUNK // L

    pltpu.sync_copy(wvec_hbm, wv)

    base_chunks = N_CHUNKS // n_workers
    extra = N_CHUNKS - base_chunks * n_workers
    my_n = base_chunks + jnp.where(wid < extra, 1, 0)

    lane = lax.iota(jnp.int32, L)
    rows_of = [g * L + lane for g in range(n_groups)]

    def chunk_body(i, _):
        base = (wid + i * n_workers) * CHUNK
        pltpu.sync_copy(src_id_hbm.at[pl.ds(base, CHUNK)], idx_s)
        pltpu.sync_copy(dst_id_hbm.at[pl.ds(base, CHUNK)], idx_d)
        cp_s = pltpu.async_copy(emb_hbm.at[idx_s], src_rows, sem_s)
        cp_d = pltpu.async_copy(emb_hbm.at[idx_d], dst_rows, sem_d)
        pltpu.sync_copy(featsp_hbm.at[pl.ds(base, CHUNK), :], feats_v)
        cp_s.wait()
        cp_d.wait()

        def dim_body(d, accs):
            dv = jnp.full((L,), 0, jnp.int32) + d
            ws = plsc.load_gather(wv, [dv])
            new = []
            for g in range(n_groups):
                s = plsc.load_gather(src_rows, [rows_of[g], dv])
                t = plsc.load_gather(dst_rows, [rows_of[g], dv])
                new.append(accs[g] + s * t * ws)
            return tuple(new)

        accs = lax.fori_loop(
            0, D_EMB_C, dim_body,
            tuple(jnp.zeros((L,), jnp.float32) for _ in range(n_groups)))

        for f in range(7):
            fv = jnp.full((L,), f, jnp.int32)
            wf = plsc.load_gather(wv, [jnp.full((L,), D_EMB_C + f, jnp.int32)])
            accs = tuple(accs[g] + plsc.load_gather(feats_v, [rows_of[g], fv]) * wf
                         for g in range(n_groups))

        for g in range(n_groups):
            out_v[pl.ds(g * L, L)] = 1.0 / (1.0 + jnp.exp(-accs[g]))

        pltpu.sync_copy(out_v, out_hbm.at[pl.ds(base, CHUNK)])
        return ()

    lax.fori_loop(0, my_n, chunk_body, ())


@jax.jit
def _run(embedding, src_id, dst_id, featsp, wvec):
    mesh = plsc.VectorSubcoreMesh(core_axis_name="c", subcore_axis_name="s")
    k = functools.partial(
        pl.kernel,
        out_type=jax.ShapeDtypeStruct((E_C,), jnp.float32),
        mesh=mesh,
        compiler_params=pltpu.CompilerParams(needs_layout_passes=False),
        scratch_types=[
            pltpu.VMEM((CHUNK,), jnp.int32),
            pltpu.VMEM((CHUNK,), jnp.int32),
            pltpu.VMEM((CHUNK, D_EMB_C), jnp.float32),
            pltpu.VMEM((CHUNK, D_EMB_C), jnp.float32),
            pltpu.VMEM((CHUNK, DF), jnp.float32),
            pltpu.VMEM((CHUNK,), jnp.float32),
            pltpu.VMEM((D_EMB_C + L,), jnp.float32),
            pltpu.SemaphoreType.DMA,
            pltpu.SemaphoreType.DMA,
        ],
    )(_sc_kernel_body)
    return k(embedding, src_id, dst_id, featsp, wvec)


def kernel(embedding, src_id, dst_id, edge_feats, W, b):
    E = src_id.shape[0]
    src32 = src_id.astype(jnp.int32)
    dst32 = dst_id.astype(jnp.int32)
    featsp = jnp.concatenate(
        [edge_feats.astype(jnp.float32),
         jnp.ones((E, 1), jnp.float32),
         jnp.zeros((E, DF - 1 - edge_feats.shape[1]), jnp.float32)], axis=1)
    w = W[:, 0].astype(jnp.float32)
    wvec = jnp.concatenate(
        [w, b.astype(jnp.float32).reshape(1), jnp.zeros((L - 1 - (w.shape[0] - D_EMB_C),), jnp.float32)])
    out = _run(embedding.astype(jnp.float32), src32, dst32, featsp, wvec)
    return out.reshape(E, 1)


# speedup vs baseline: 1.0416x; 1.0416x over previous
import functools

import jax
import jax.numpy as jnp
from jax import lax
from jax.experimental import pallas as pl
from jax.experimental.pallas import tpu as pltpu
from jax.experimental.pallas import tpu_sc as plsc

N_NODES_C = 10000
D_EMB_C = 128
E_C = 320000
CHUNK = 128
N_CHUNKS = E_C // CHUNK
L = 16
DF = 16
NBUF = 2


def _sc_kernel_body(emb_hbm, src_id_hbm, dst_id_hbm, featsp_hbm, wvec_hbm,
                    out_hbm,
                    idx_s, idx_d, src_rows, dst_rows, feats_v, out_v, wv,
                    sem_is, sem_id, sem_gs, sem_gd, sem_ft, sem_out, sem_w):
    nc = plsc.get_sparse_core_info().num_cores
    wid = lax.axis_index("s") * nc + lax.axis_index("c")
    n_workers = 32
    n_groups = CHUNK // L

    pltpu.async_copy(wvec_hbm, wv, sem_w).wait()

    base_chunks = N_CHUNKS // n_workers
    extra = N_CHUNKS - base_chunks * n_workers
    my_n = base_chunks + jnp.where(wid < extra, 1, 0)

    lane = lax.iota(jnp.int32, L)
    rows_of = [g * L + lane for g in range(n_groups)]

    def ebase(c):
        return (wid + c * n_workers) * CHUNK

    def issue_idx(c, b):
        pltpu.async_copy(src_id_hbm.at[pl.ds(ebase(c), CHUNK)], idx_s[b], sem_is[b])
        pltpu.async_copy(dst_id_hbm.at[pl.ds(ebase(c), CHUNK)], idx_d[b], sem_id[b])

    def wait_idx(b):
        pltpu.make_async_copy(src_id_hbm.at[pl.ds(0, CHUNK)], idx_s[b], sem_is[b]).wait()
        pltpu.make_async_copy(dst_id_hbm.at[pl.ds(0, CHUNK)], idx_d[b], sem_id[b]).wait()

    def issue_gathers(c, b):
        pltpu.async_copy(emb_hbm.at[idx_s[b]], src_rows[b], sem_gs[b])
        pltpu.async_copy(emb_hbm.at[idx_d[b]], dst_rows[b], sem_gd[b])
        pltpu.async_copy(featsp_hbm.at[pl.ds(ebase(c), CHUNK), :], feats_v[b], sem_ft[b])

    def wait_gathers(b):
        pltpu.make_async_copy(emb_hbm.at[idx_s[b]], src_rows[b], sem_gs[b]).wait()
        pltpu.make_async_copy(emb_hbm.at[idx_d[b]], dst_rows[b], sem_gd[b]).wait()
        pltpu.make_async_copy(featsp_hbm.at[pl.ds(0, CHUNK), :], feats_v[b], sem_ft[b]).wait()

    def compute(b):
        zero = tuple(jnp.zeros((L,), jnp.float32) for _ in range(n_groups))

        @plsc.parallel_loop(0, D_EMB_C, 1, unroll=8, carry=zero)
        def accs(d, accs_in):
            dv = jnp.full((L,), 0, jnp.int32) + d
            ws = plsc.load_gather(wv, [dv])
            new = []
            for g in range(n_groups):
                s = plsc.load_gather(src_rows[b], [rows_of[g], dv])
                t = plsc.load_gather(dst_rows[b], [rows_of[g], dv])
                new.append(accs_in[g] + s * t * ws)
            return tuple(new)

        for f in range(7):
            fv = jnp.full((L,), f, jnp.int32)
            wf = plsc.load_gather(wv, [jnp.full((L,), D_EMB_C + f, jnp.int32)])
            accs = tuple(accs[g] + plsc.load_gather(feats_v[b], [rows_of[g], fv]) * wf
                         for g in range(n_groups))

        for g in range(n_groups):
            out_v[b][pl.ds(g * L, L)] = 1.0 / (1.0 + jnp.exp(-accs[g]))

    def issue_out(c, b):
        pltpu.async_copy(out_v[b], out_hbm.at[pl.ds(ebase(c), CHUNK)], sem_out[b])

    def wait_out(b):
        pltpu.make_async_copy(out_v[b], out_hbm.at[pl.ds(0, CHUNK)], sem_out[b]).wait()

    issue_idx(0, 0)
    issue_idx(1, 1)
    wait_idx(0)
    issue_gathers(0, 0)

    def outer(i2, _):
        for bpar in range(NBUF):
            c = i2 * NBUF + bpar
            b = bpar

            @pl.when(c < my_n)
            def _():
                nb = 1 - b
                wait_gathers(b)

                @pl.when(c + 1 < my_n)
                def _():
                    wait_idx(nb)
                    issue_gathers(c + 1, nb)

                @pl.when(c + 2 < my_n)
                def _():
                    issue_idx(c + 2, b)

                @pl.when(c >= NBUF)
                def _():
                    wait_out(b)

                compute(b)
                issue_out(c, b)
        return ()

    lax.fori_loop(0, (base_chunks + 1 + NBUF - 1) // NBUF, outer, ())
    for b in range(NBUF):
        wait_out(b)


@jax.jit
def _run(embedding, src_id, dst_id, featsp, wvec):
    mesh = plsc.VectorSubcoreMesh(core_axis_name="c", subcore_axis_name="s")
    vm = pltpu.VMEM
    k = functools.partial(
        pl.kernel,
        out_type=jax.ShapeDtypeStruct((E_C,), jnp.float32),
        mesh=mesh,
        compiler_params=pltpu.CompilerParams(needs_layout_passes=False),
        scratch_types=[
            [vm((CHUNK,), jnp.int32) for _ in range(NBUF)],
            [vm((CHUNK,), jnp.int32) for _ in range(NBUF)],
            [vm((CHUNK, D_EMB_C), jnp.float32) for _ in range(NBUF)],
            [vm((CHUNK, D_EMB_C), jnp.float32) for _ in range(NBUF)],
            [vm((CHUNK, DF), jnp.float32) for _ in range(NBUF)],
            [vm((CHUNK,), jnp.float32) for _ in range(NBUF)],
            vm((D_EMB_C + L,), jnp.float32),
            [pltpu.SemaphoreType.DMA for _ in range(NBUF)],
            [pltpu.SemaphoreType.DMA for _ in range(NBUF)],
            [pltpu.SemaphoreType.DMA for _ in range(NBUF)],
            [pltpu.SemaphoreType.DMA for _ in range(NBUF)],
            [pltpu.SemaphoreType.DMA for _ in range(NBUF)],
            [pltpu.SemaphoreType.DMA for _ in range(NBUF)],
            pltpu.SemaphoreType.DMA,
        ],
    )(_sc_kernel_body)
    return k(embedding, src_id, dst_id, featsp, wvec)


def kernel(embedding, src_id, dst_id, edge_feats, W, b):
    E = src_id.shape[0]
    src32 = src_id.astype(jnp.int32)
    dst32 = dst_id.astype(jnp.int32)
    featsp = jnp.concatenate(
        [edge_feats.astype(jnp.float32),
         jnp.ones((E, 1), jnp.float32),
         jnp.zeros((E, DF - 1 - edge_feats.shape[1]), jnp.float32)], axis=1)
    w = W[:, 0].astype(jnp.float32)
    wvec = jnp.concatenate(
        [w, b.astype(jnp.float32).reshape(1),
         jnp.zeros((L - 1 - (w.shape[0] - D_EMB_C),), jnp.float32)])
    out = _run(embedding.astype(jnp.float32), src32, dst32, featsp, wvec)
    return out.reshape(E, 1)


# speedup vs baseline: 3.4019x; 3.2659x over previous
import functools

import jax
import jax.numpy as jnp
from jax import lax
from jax.experimental import pallas as pl
from jax.experimental.pallas import tpu as pltpu
from jax.experimental.pallas import tpu_sc as plsc

N_NODES_C = 10000
D_EMB_C = 128
E_C = 320000
CHUNK = 128
N_CHUNKS = E_C // CHUNK
L = 16
DF = 16
NBUF = 2


def _sc_kernel_body(emb_hbm, src_id_hbm, dst_id_hbm, featsp_hbm, wvec_hbm,
                    out_hbm,
                    idx_s, idx_d, src_rows, dst_rows, feats_v, out_v, wv,
                    sem_is, sem_id, sem_gs, sem_gd, sem_ft, sem_out, sem_w):
    nc = plsc.get_sparse_core_info().num_cores
    wid = lax.axis_index("s") * nc + lax.axis_index("c")
    n_workers = 32
    n_groups = CHUNK // L

    pltpu.async_copy(wvec_hbm, wv, sem_w).wait()

    base_chunks = N_CHUNKS // n_workers
    extra = N_CHUNKS - base_chunks * n_workers
    my_n = base_chunks + jnp.where(wid < extra, 1, 0)

    lane = lax.iota(jnp.int32, L)
    rows_of = [g * L + lane for g in range(n_groups)]

    def ebase(c):
        return (wid + c * n_workers) * CHUNK

    def issue_idx(c, b):
        pltpu.async_copy(src_id_hbm.at[pl.ds(ebase(c), CHUNK)], idx_s[b], sem_is[b])
        pltpu.async_copy(dst_id_hbm.at[pl.ds(ebase(c), CHUNK)], idx_d[b], sem_id[b])

    def wait_idx(b):
        pltpu.make_async_copy(src_id_hbm.at[pl.ds(0, CHUNK)], idx_s[b], sem_is[b]).wait()
        pltpu.make_async_copy(dst_id_hbm.at[pl.ds(0, CHUNK)], idx_d[b], sem_id[b]).wait()

    def issue_gathers(c, b):
        pltpu.async_copy(emb_hbm.at[idx_s[b]], src_rows[b], sem_gs[b])
        pltpu.async_copy(emb_hbm.at[idx_d[b]], dst_rows[b], sem_gd[b])
        pltpu.async_copy(featsp_hbm.at[pl.ds(ebase(c), CHUNK), :], feats_v[b], sem_ft[b])

    def wait_gathers(b):
        pltpu.make_async_copy(emb_hbm.at[idx_s[b]], src_rows[b], sem_gs[b]).wait()
        pltpu.make_async_copy(emb_hbm.at[idx_d[b]], dst_rows[b], sem_gd[b]).wait()
        pltpu.make_async_copy(featsp_hbm.at[pl.ds(0, CHUNK), :], feats_v[b], sem_ft[b]).wait()

    def compute(b):
        zero = tuple(jnp.zeros((L,), jnp.float32) for _ in range(n_groups))

        @plsc.parallel_loop(0, D_EMB_C, 1, unroll=8, carry=zero)
        def accs(d, accs_in):
            col = (jnp.full((L,), 0, jnp.int32) + d + lane) & (D_EMB_C - 1)
            ws = plsc.load_gather(wv, [col])
            new = []
            for g in range(n_groups):
                s = plsc.load_gather(src_rows[b], [rows_of[g], col])
                t = plsc.load_gather(dst_rows[b], [rows_of[g], col])
                new.append(accs_in[g] + s * t * ws)
            return tuple(new)

        for f in range(DF):
            col = (jnp.full((L,), f, jnp.int32) + lane) & (DF - 1)
            wf = plsc.load_gather(wv, [col + D_EMB_C])
            accs = tuple(accs[g] + plsc.load_gather(feats_v[b], [rows_of[g], col]) * wf
                         for g in range(n_groups))

        for g in range(n_groups):
            out_v[b][pl.ds(g * L, L)] = 1.0 / (1.0 + jnp.exp(-accs[g]))

    def issue_out(c, b):
        pltpu.async_copy(out_v[b], out_hbm.at[pl.ds(ebase(c), CHUNK)], sem_out[b])

    def wait_out(b):
        pltpu.make_async_copy(out_v[b], out_hbm.at[pl.ds(0, CHUNK)], sem_out[b]).wait()

    issue_idx(0, 0)
    issue_idx(1, 1)
    wait_idx(0)
    issue_gathers(0, 0)

    def outer(i2, _):
        for bpar in range(NBUF):
            c = i2 * NBUF + bpar
            b = bpar

            @pl.when(c < my_n)
            def _():
                nb = 1 - b
                wait_gathers(b)

                @pl.when(c + 1 < my_n)
                def _():
                    wait_idx(nb)
                    issue_gathers(c + 1, nb)

                @pl.when(c + 2 < my_n)
                def _():
                    issue_idx(c + 2, b)

                @pl.when(c >= NBUF)
                def _():
                    wait_out(b)

                compute(b)
                issue_out(c, b)
        return ()

    lax.fori_loop(0, (base_chunks + 1 + NBUF - 1) // NBUF, outer, ())
    for b in range(NBUF):
        wait_out(b)


@jax.jit
def _run(embedding, src_id, dst_id, featsp, wvec):
    mesh = plsc.VectorSubcoreMesh(core_axis_name="c", subcore_axis_name="s")
    vm = pltpu.VMEM
    k = functools.partial(
        pl.kernel,
        out_type=jax.ShapeDtypeStruct((E_C,), jnp.float32),
        mesh=mesh,
        compiler_params=pltpu.CompilerParams(needs_layout_passes=False),
        scratch_types=[
            [vm((CHUNK,), jnp.int32) for _ in range(NBUF)],
            [vm((CHUNK,), jnp.int32) for _ in range(NBUF)],
            [vm((CHUNK, D_EMB_C), jnp.float32) for _ in range(NBUF)],
            [vm((CHUNK, D_EMB_C), jnp.float32) for _ in range(NBUF)],
            [vm((CHUNK, DF), jnp.float32) for _ in range(NBUF)],
            [vm((CHUNK,), jnp.float32) for _ in range(NBUF)],
            vm((D_EMB_C + L,), jnp.float32),
            [pltpu.SemaphoreType.DMA for _ in range(NBUF)],
            [pltpu.SemaphoreType.DMA for _ in range(NBUF)],
            [pltpu.SemaphoreType.DMA for _ in range(NBUF)],
            [pltpu.SemaphoreType.DMA for _ in range(NBUF)],
            [pltpu.SemaphoreType.DMA for _ in range(NBUF)],
            [pltpu.SemaphoreType.DMA for _ in range(NBUF)],
            pltpu.SemaphoreType.DMA,
        ],
    )(_sc_kernel_body)
    return k(embedding, src_id, dst_id, featsp, wvec)


def kernel(embedding, src_id, dst_id, edge_feats, W, b):
    E = src_id.shape[0]
    src32 = src_id.astype(jnp.int32)
    dst32 = dst_id.astype(jnp.int32)
    featsp = jnp.concatenate(
        [edge_feats.astype(jnp.float32),
         jnp.ones((E, 1), jnp.float32),
         jnp.zeros((E, DF - 1 - edge_feats.shape[1]), jnp.float32)], axis=1)
    w = W[:, 0].astype(jnp.float32)
    wvec = jnp.concatenate(
        [w, b.astype(jnp.float32).reshape(1),
         jnp.zeros((L - 1 - (w.shape[0] - D_EMB_C),), jnp.float32)])
    out = _run(embedding.astype(jnp.float32), src32, dst32, featsp, wvec)
    return out.reshape(E, 1)


# speedup vs baseline: 5.0429x; 1.4824x over previous
import functools

import jax
import jax.numpy as jnp
from jax import lax
from jax.experimental import pallas as pl
from jax.experimental.pallas import tpu as pltpu
from jax.experimental.pallas import tpu_sc as plsc

N_NODES_C = 10000
D_EMB_C = 128
E_C = 320000
CHUNK = 32
N_CHUNKS = E_C // CHUNK
L = 16
DF = 16
NBUF = 2


def _sc_kernel_body(emb_hbm, src_id_hbm, dst_id_hbm, featsp_hbm, wvec_hbm,
                    out_hbm,
                    idx_s, idx_d, src_rows, dst_rows, feats_v, out_v, wv,
                    tbl_sh,
                    sem_is, sem_id, sem_gs, sem_gd, sem_ft, sem_out, sem_w):
    nc = plsc.get_sparse_core_info().num_cores
    sid = lax.axis_index("s")
    wid = sid * nc + lax.axis_index("c")
    n_workers = 32
    n_groups = CHUNK // L

    pltpu.async_copy(wvec_hbm, wv, sem_w).wait()

    rpt = 624
    pltpu.sync_copy(emb_hbm.at[pl.ds(sid * rpt, rpt), :],
                    tbl_sh.at[pl.ds(sid * rpt, rpt), :])

    @pl.when(sid == 0)
    def _():
        pltpu.sync_copy(emb_hbm.at[pl.ds(16 * rpt, N_NODES_C - 16 * rpt), :],
                        tbl_sh.at[pl.ds(16 * rpt, N_NODES_C - 16 * rpt), :])

    plsc.subcore_barrier()

    base_chunks = N_CHUNKS // n_workers
    extra = N_CHUNKS - base_chunks * n_workers
    my_n = base_chunks + jnp.where(wid < extra, 1, 0)

    lane = lax.iota(jnp.int32, L)
    rows_of = [g * L + lane for g in range(n_groups)]

    def ebase(c):
        return (wid + c * n_workers) * CHUNK

    def issue_idx(c, b):
        pltpu.async_copy(src_id_hbm.at[pl.ds(ebase(c), CHUNK)], idx_s[b], sem_is[b])
        pltpu.async_copy(dst_id_hbm.at[pl.ds(ebase(c), CHUNK)], idx_d[b], sem_id[b])

    def wait_idx(b):
        pltpu.make_async_copy(src_id_hbm.at[pl.ds(0, CHUNK)], idx_s[b], sem_is[b]).wait()
        pltpu.make_async_copy(dst_id_hbm.at[pl.ds(0, CHUNK)], idx_d[b], sem_id[b]).wait()

    def issue_gathers(c, b):
        pltpu.async_copy(tbl_sh.at[idx_s[b]], src_rows[b], sem_gs[b])
        pltpu.async_copy(tbl_sh.at[idx_d[b]], dst_rows[b], sem_gd[b])
        pltpu.async_copy(featsp_hbm.at[pl.ds(ebase(c), CHUNK), :], feats_v[b], sem_ft[b])

    def wait_gathers(b):
        pltpu.make_async_copy(tbl_sh.at[idx_s[b]], src_rows[b], sem_gs[b]).wait()
        pltpu.make_async_copy(tbl_sh.at[idx_d[b]], dst_rows[b], sem_gd[b]).wait()
        pltpu.make_async_copy(featsp_hbm.at[pl.ds(0, CHUNK), :], feats_v[b], sem_ft[b]).wait()

    def compute(b):
        zero = tuple(jnp.zeros((L,), jnp.float32) for _ in range(n_groups))

        @plsc.parallel_loop(0, D_EMB_C, 1, unroll=8, carry=zero)
        def accs(d, accs_in):
            col = (jnp.full((L,), 0, jnp.int32) + d + lane) & (D_EMB_C - 1)
            ws = plsc.load_gather(wv, [col])
            new = []
            for g in range(n_groups):
                s = plsc.load_gather(src_rows[b], [rows_of[g], col])
                t = plsc.load_gather(dst_rows[b], [rows_of[g], col])
                new.append(accs_in[g] + s * t * ws)
            return tuple(new)

        for f in range(DF):
            col = (jnp.full((L,), f, jnp.int32) + lane) & (DF - 1)
            wf = plsc.load_gather(wv, [col + D_EMB_C])
            accs = tuple(accs[g] + plsc.load_gather(feats_v[b], [rows_of[g], col]) * wf
                         for g in range(n_groups))

        for g in range(n_groups):
            out_v[b][pl.ds(g * L, L)] = 1.0 / (1.0 + jnp.exp(-accs[g]))

    def issue_out(c, b):
        pltpu.async_copy(out_v[b], out_hbm.at[pl.ds(ebase(c), CHUNK)], sem_out[b])

    def wait_out(b):
        pltpu.make_async_copy(out_v[b], out_hbm.at[pl.ds(0, CHUNK)], sem_out[b]).wait()

    issue_idx(0, 0)
    issue_idx(1, 1)
    wait_idx(0)
    issue_gathers(0, 0)

    def outer(i2, _):
        for bpar in range(NBUF):
            c = i2 * NBUF + bpar
            b = bpar

            @pl.when(c < my_n)
            def _():
                nb = 1 - b
                wait_gathers(b)

                @pl.when(c + 1 < my_n)
                def _():
                    wait_idx(nb)
                    issue_gathers(c + 1, nb)

                @pl.when(c + 2 < my_n)
                def _():
                    issue_idx(c + 2, b)

                @pl.when(c >= NBUF)
                def _():
                    wait_out(b)

                compute(b)
                issue_out(c, b)
        return ()

    lax.fori_loop(0, (base_chunks + 1 + NBUF - 1) // NBUF, outer, ())
    for b in range(NBUF):
        wait_out(b)


@jax.jit
def _run(embedding, src_id, dst_id, featsp, wvec):
    mesh = plsc.VectorSubcoreMesh(core_axis_name="c", subcore_axis_name="s")
    vm = pltpu.VMEM
    k = functools.partial(
        pl.kernel,
        out_type=jax.ShapeDtypeStruct((E_C,), jnp.float32),
        mesh=mesh,
        compiler_params=pltpu.CompilerParams(needs_layout_passes=False),
        scratch_types=[
            [vm((CHUNK,), jnp.int32) for _ in range(NBUF)],
            [vm((CHUNK,), jnp.int32) for _ in range(NBUF)],
            [vm((CHUNK, D_EMB_C), jnp.float32) for _ in range(NBUF)],
            [vm((CHUNK, D_EMB_C), jnp.float32) for _ in range(NBUF)],
            [vm((CHUNK, DF), jnp.float32) for _ in range(NBUF)],
            [vm((CHUNK,), jnp.float32) for _ in range(NBUF)],
            vm((D_EMB_C + L,), jnp.float32),
            pltpu.VMEM_SHARED((N_NODES_C, D_EMB_C), jnp.float32),
            [pltpu.SemaphoreType.DMA for _ in range(NBUF)],
            [pltpu.SemaphoreType.DMA for _ in range(NBUF)],
            [pltpu.SemaphoreType.DMA for _ in range(NBUF)],
            [pltpu.SemaphoreType.DMA for _ in range(NBUF)],
            [pltpu.SemaphoreType.DMA for _ in range(NBUF)],
            [pltpu.SemaphoreType.DMA for _ in range(NBUF)],
            pltpu.SemaphoreType.DMA,
        ],
    )(_sc_kernel_body)
    return k(embedding, src_id, dst_id, featsp, wvec)


def kernel(embedding, src_id, dst_id, edge_feats, W, b):
    E = src_id.shape[0]
    src32 = src_id.astype(jnp.int32)
    dst32 = dst_id.astype(jnp.int32)
    featsp = jnp.concatenate(
        [edge_feats.astype(jnp.float32),
         jnp.ones((E, 1), jnp.float32),
         jnp.zeros((E, DF - 1 - edge_feats.shape[1]), jnp.float32)], axis=1)
    w = W[:, 0].astype(jnp.float32)
    wvec = jnp.concatenate(
        [w, b.astype(jnp.float32).reshape(1),
         jnp.zeros((L - 1 - (w.shape[0] - D_EMB_C),), jnp.float32)])
    out = _run(embedding.astype(jnp.float32), src32, dst32, featsp, wvec)
    return out.reshape(E, 1)
